# baseline (speedup 1.0000x reference)
import functools

import jax
import jax.numpy as jnp
from jax import lax
from jax.experimental import pallas as pl
from jax.experimental.pallas import tpu as pltpu
from jax.experimental.pallas import tpu_sc as plsc
from jax.experimental.layout import Layout, with_layout_constraint

_B, _S, _D = 4, 4096, 2048
_SURV = _S - (_S + 2) // 3
_NC, _NS = 2, 16
_TROWS = _B * _S * 16
_OROWS = _B * _SURV * 16
_FULL = 42


@functools.partial(
    pl.kernel,
    mesh=plsc.VectorSubcoreMesh(core_axis_name="c", subcore_axis_name="s"),
    out_type=jax.ShapeDtypeStruct((_OROWS, 128), jnp.float32),
    scratch_types=[
        pltpu.VMEM((43, 128), jnp.int32),
        pltpu.VMEM((128, 128), jnp.float32),
        pltpu.VMEM((128, 128), jnp.float32),
        pltpu.VMEM((128, 128), jnp.float32),
        pltpu.VMEM((128, 128), jnp.float32),
        pltpu.VMEM((64, 128), jnp.float32),
        pltpu.SemaphoreType.DMA,
        pltpu.SemaphoreType.DMA,
        pltpu.SemaphoreType.DMA,
        pltpu.SemaphoreType.DMA,
        pltpu.SemaphoreType.DMA,
        pltpu.SemaphoreType.DMA,
        pltpu.SemaphoreType.DMA,
        pltpu.SemaphoreType.DMA,
        pltpu.SemaphoreType.DMA,
    ],
)
def _prune(table, out, idx_v, buf0, buf1, buf2, buf3, tb,
           g0, g1, g2, g3, w0, w1, w2, w3, tsem):
    wid = lax.axis_index("s") * _NC + lax.axis_index("c")
    tok0 = wid * 85 + jnp.minimum(wid, 10)
    u0 = tok0 * 64
    lanes = lax.iota(jnp.int32, 16)

    def gen_idx(t):
        for k in range(8):
            u = (tok0 + 2 * t) * 64 + k * 16 + lanes
            j = u >> 6
            src = j + (j >> 1) + 1
            v = ((u & 3) * 65536 + (src >> 3) * 128
                 + ((u >> 2) & 15) * 8 + (src & 7))
            idx_v[t, pl.ds(k * 16, 16)] = jnp.minimum(v, _TROWS - 1)

    bufs = (buf0, buf1, buf2, buf3)
    gsems = (g0, g1, g2, g3)
    wsems = (w0, w1, w2, w3)

    for t in range(4):
        gen_idx(t)
    copies = [
        pltpu.async_copy(table.at[idx_v.at[t]], bufs[t], gsems[t])
        for t in range(4)
    ]

    def idx_body(t, carry):
        gen_idx(t)
        return carry

    lax.fori_loop(4, 43, idx_body, 0, unroll=False)

    wcopies = [None, None, None, None]
    for t in range(_FULL):
        s = t % 4
        nxt = t + 2
        if t >= 2 and nxt < _FULL:
            sp = nxt % 4
            if wcopies[sp] is not None:
                wcopies[sp].wait()
            copies[sp] = pltpu.async_copy(
                table.at[idx_v.at[nxt]], bufs[sp], gsems[sp])
        copies[s].wait()
        wcopies[s] = pltpu.async_copy(
            bufs[s], out.at[pl.ds(u0 + t * 128, 128)], wsems[s])
    for s in range(4):
        if wcopies[s] is not None:
            wcopies[s].wait()

    @pl.when(wid < 10)
    def _last_full():
        pltpu.async_copy(table.at[idx_v.at[42]], buf0, tsem).wait()
        pltpu.sync_copy(buf0, out.at[pl.ds(u0 + _FULL * 128, 128)])

    @pl.when(wid >= 10)
    def _last_half():
        pltpu.async_copy(table.at[idx_v.at[42, pl.ds(0, 64)]], tb, tsem).wait()
        pltpu.sync_copy(tb, out.at[pl.ds(u0 + _FULL * 128, 64)])


def kernel(hidden_states):
    t5 = hidden_states.reshape(_B, _S // 8, 8, 16, 128)
    t5 = jnp.transpose(t5, (0, 1, 3, 2, 4))
    table = t5.reshape(_TROWS, 128)
    flat = _prune(table)
    v = flat.reshape(_SURV, 16, 4, 128)
    v = with_layout_constraint(
        v, Layout(major_to_minor=(0, 1, 2, 3), tiling=((4, 128),)))
    t = jnp.transpose(v, (2, 0, 1, 3))
    t = with_layout_constraint(
        t, Layout(major_to_minor=(1, 2, 0, 3), tiling=((4, 128),)))
    return t.reshape(_B, _SURV, _D)


# speedup vs baseline: 1.0055x; 1.0055x over previous
import functools

import jax
import jax.numpy as jnp
from jax import lax
from jax.experimental import pallas as pl
from jax.experimental.pallas import tpu as pltpu
from jax.experimental.pallas import tpu_sc as plsc
from jax.experimental.layout import Layout, with_layout_constraint

_B, _S, _D = 4, 4096, 2048
_SURV = _S - (_S + 2) // 3
_NC, _NS = 2, 16
_TROWS = _B * _S * 16
_OROWS = _B * _SURV * 16
_FULL = 42


@functools.partial(
    pl.kernel,
    mesh=plsc.VectorSubcoreMesh(core_axis_name="c", subcore_axis_name="s"),
    out_type=jax.ShapeDtypeStruct((_OROWS, 128), jnp.float32),
    scratch_types=[
        pltpu.VMEM((43, 128), jnp.int32),
        pltpu.VMEM((128, 128), jnp.float32),
        pltpu.VMEM((128, 128), jnp.float32),
        pltpu.VMEM((128, 128), jnp.float32),
        pltpu.VMEM((64, 128), jnp.float32),
        pltpu.SemaphoreType.DMA,
        pltpu.SemaphoreType.DMA,
        pltpu.SemaphoreType.DMA,
        pltpu.SemaphoreType.DMA,
        pltpu.SemaphoreType.DMA,
        pltpu.SemaphoreType.DMA,
        pltpu.SemaphoreType.DMA,
    ],
)
def _prune(table, out, idx_v, buf0, buf1, buf2, tb,
           g0, g1, g2, w0, w1, w2, tsem):
    wid = lax.axis_index("s") * _NC + lax.axis_index("c")
    tok0 = wid * 85 + jnp.minimum(wid, 10)
    u0 = tok0 * 64
    lanes = lax.iota(jnp.int32, 16)

    def gen_idx(t):
        for k in range(8):
            u = (tok0 + 2 * t) * 64 + k * 16 + lanes
            j = u >> 6
            src = j + (j >> 1) + 1
            v = ((u & 3) * 65536 + (src >> 3) * 128
                 + ((u >> 2) & 15) * 8 + (src & 7))
            idx_v[t, pl.ds(k * 16, 16)] = jnp.minimum(v, _TROWS - 1)

    bufs = (buf0, buf1, buf2)
    gsems = (g0, g1, g2)
    wsems = (w0, w1, w2)

    for t in range(3):
        gen_idx(t)
    copies = [
        pltpu.async_copy(table.at[idx_v.at[t]], bufs[t], gsems[t])
        for t in range(3)
    ]

    def idx_body(t, carry):
        gen_idx(t)
        return carry

    lax.fori_loop(3, 43, idx_body, 0, unroll=False)

    wcopies = [None, None, None]
    for t in range(_FULL):
        s = t % 3
        nxt = t + 2
        if t >= 1 and nxt < _FULL:
            sp = nxt % 3
            if wcopies[sp] is not None:
                wcopies[sp].wait()
            copies[sp] = pltpu.async_copy(
                table.at[idx_v.at[nxt]], bufs[sp], gsems[sp])
        copies[s].wait()
        wcopies[s] = pltpu.async_copy(
            bufs[s], out.at[pl.ds(u0 + t * 128, 128)], wsems[s])
    for s in range(3):
        if wcopies[s] is not None:
            wcopies[s].wait()

    @pl.when(wid < 10)
    def _last_full():
        pltpu.async_copy(table.at[idx_v.at[42]], buf0, tsem).wait()
        pltpu.sync_copy(buf0, out.at[pl.ds(u0 + _FULL * 128, 128)])

    @pl.when(wid >= 10)
    def _last_half():
        pltpu.async_copy(table.at[idx_v.at[42, pl.ds(0, 64)]], tb, tsem).wait()
        pltpu.sync_copy(tb, out.at[pl.ds(u0 + _FULL * 128, 64)])


def kernel(hidden_states):
    t5 = hidden_states.reshape(_B, _S // 8, 8, 16, 128)
    t5 = jnp.transpose(t5, (0, 1, 3, 2, 4))
    table = t5.reshape(_TROWS, 128)
    flat = _prune(table)
    v = flat.reshape(_SURV, 16, 4, 128)
    v = with_layout_constraint(
        v, Layout(major_to_minor=(0, 1, 2, 3), tiling=((4, 128),)))
    t = jnp.transpose(v, (2, 0, 1, 3))
    t = with_layout_constraint(
        t, Layout(major_to_minor=(1, 2, 0, 3), tiling=((4, 128),)))
    return t.reshape(_B, _SURV, _D)


# speedup vs baseline: 1.0174x; 1.0118x over previous
import functools

import jax
import jax.numpy as jnp
from jax import lax
from jax.experimental import pallas as pl
from jax.experimental.pallas import tpu as pltpu
from jax.experimental.pallas import tpu_sc as plsc
from jax.experimental.layout import Layout, with_layout_constraint

_B, _S, _D = 4, 4096, 2048
_SURV = _S - (_S + 2) // 3
_NC, _NS = 2, 16
_TROWS = _B * _S * 16
_OROWS = _B * _SURV * 16
_FULL = 42


@functools.partial(
    pl.kernel,
    mesh=plsc.VectorSubcoreMesh(core_axis_name="c", subcore_axis_name="s"),
    out_type=jax.ShapeDtypeStruct((_OROWS, 128), jnp.float32),
    scratch_types=[
        pltpu.VMEM((43, 128), jnp.int32),
        pltpu.VMEM((128, 128), jnp.float32),
        pltpu.VMEM((128, 128), jnp.float32),
        pltpu.VMEM((128, 128), jnp.float32),
        pltpu.VMEM((64, 128), jnp.float32),
        pltpu.SemaphoreType.DMA,
        pltpu.SemaphoreType.DMA,
        pltpu.SemaphoreType.DMA,
        pltpu.SemaphoreType.DMA,
        pltpu.SemaphoreType.DMA,
        pltpu.SemaphoreType.DMA,
        pltpu.SemaphoreType.DMA,
    ],
)
def _prune(table, out, idx_v, buf0, buf1, buf2, tb,
           g0, g1, g2, w0, w1, w2, tsem):
    wid = lax.axis_index("s") * _NC + lax.axis_index("c")
    tok0 = wid * 85 + jnp.minimum(wid, 10)
    u0 = tok0 * 64
    lanes = lax.iota(jnp.int32, 16)

    def gen_idx(t):
        for k in range(8):
            u = (tok0 + 2 * t) * 64 + k * 16 + lanes
            j = u >> 6
            src = j + (j >> 1) + 1
            v = ((u & 3) * 65536 + (src >> 3) * 128
                 + ((u >> 2) & 15) * 8 + (src & 7))
            idx_v[t, pl.ds(k * 16, 16)] = jnp.minimum(v, _TROWS - 1)

    bufs = (buf0, buf1, buf2)
    gsems = (g0, g1, g2)
    wsems = (w0, w1, w2)

    for t in range(3):
        gen_idx(t)
    copies = [
        pltpu.async_copy(table.at[idx_v.at[t]], bufs[t], gsems[t])
        for t in range(3)
    ]

    def idx_body(t, carry):
        gen_idx(t)
        return carry

    lax.fori_loop(3, 43, idx_body, 0, unroll=False)

    def pipe_body(i, carry):
        for k in range(3):
            t = 3 * i + k
            sp = (k + 2) % 3

            @pl.when(jnp.logical_and(t >= 1, t < 40))
            def _fire():
                pltpu.make_async_copy(
                    bufs[sp], out.at[pl.ds(u0, 128)], wsems[sp]).wait()
                pltpu.async_copy(
                    table.at[idx_v.at[t + 2]], bufs[sp], gsems[sp])

            pltpu.make_async_copy(
                table.at[idx_v.at[0]], bufs[k], gsems[k]).wait()
            pltpu.async_copy(
                bufs[k], out.at[pl.ds(u0 + t * 128, 128)], wsems[k])
        return carry

    lax.fori_loop(0, _FULL // 3, pipe_body, 0, unroll=False)
    for s in range(3):
        pltpu.make_async_copy(bufs[s], out.at[pl.ds(u0, 128)], wsems[s]).wait()

    @pl.when(wid < 10)
    def _last_full():
        pltpu.async_copy(table.at[idx_v.at[42]], buf0, tsem).wait()
        pltpu.sync_copy(buf0, out.at[pl.ds(u0 + _FULL * 128, 128)])

    @pl.when(wid >= 10)
    def _last_half():
        pltpu.async_copy(table.at[idx_v.at[42, pl.ds(0, 64)]], tb, tsem).wait()
        pltpu.sync_copy(tb, out.at[pl.ds(u0 + _FULL * 128, 64)])


def kernel(hidden_states):
    t5 = hidden_states.reshape(_B, _S // 8, 8, 16, 128)
    t5 = jnp.transpose(t5, (0, 1, 3, 2, 4))
    table = t5.reshape(_TROWS, 128)
    flat = _prune(table)
    v = flat.reshape(_SURV, 16, 4, 128)
    v = with_layout_constraint(
        v, Layout(major_to_minor=(0, 1, 2, 3), tiling=((4, 128),)))
    t = jnp.transpose(v, (2, 0, 1, 3))
    t = with_layout_constraint(
        t, Layout(major_to_minor=(1, 2, 0, 3), tiling=((4, 128),)))
    return t.reshape(_B, _SURV, _D)
